# baseline (speedup 1.0000x reference)
import functools

import jax
import jax.numpy as jnp
from jax import lax
from jax.experimental import pallas as pl
from jax.experimental.pallas import tpu as pltpu
from jax.experimental.pallas import tpu_sc as plsc

VOCAB = 1000000
DIM = 64
B = 16384
NEG = 20

NC = 2
NS = 16
L = 16
NW = NC * NS
BPW = B // NW
C = 32
NCHUNK = BPW // C
NIDX_ROWS = C * NEG // 128

HALF = 1 << 19
RPK_ROWS = 4096
RPK_GRID = HALF // RPK_ROWS

_SC_MESH = plsc.VectorSubcoreMesh(core_axis_name="c", subcore_axis_name="s")
_SC_PARAMS = pltpu.CompilerParams(use_tc_tiling_on_sc=False)


def _repack_body(t1_ref, t2_ref, o_ref):
    i0 = lax.broadcasted_iota(jnp.int32, (DIM, 128), 0)
    i1 = lax.broadcasted_iota(jnp.int32, (DIM, 128), 1)
    w1 = (i1 == i0).astype(jnp.float32)
    w2 = (i1 == i0 + DIM).astype(jnp.float32)
    dn = (((0,), (0,)), ((), ()))
    o_ref[...] = (
        lax.dot_general(t1_ref[...], w1, dn,
                        preferred_element_type=jnp.float32)
        + lax.dot_general(t2_ref[...], w2, dn,
                          preferred_element_type=jnp.float32))


def _repack(TT):
    nblk = pl.cdiv(VOCAB, RPK_ROWS)
    lo = pl.BlockSpec((DIM, RPK_ROWS), lambda g: (0, g))
    hi = pl.BlockSpec((DIM, RPK_ROWS),
                      lambda g: (0, jnp.minimum(g + RPK_GRID, nblk - 1)))
    spec_out = pl.BlockSpec((RPK_ROWS, 128), lambda g: (g, 0))
    return pl.pallas_call(
        _repack_body,
        grid=(RPK_GRID,),
        in_specs=[lo, hi],
        out_specs=spec_out,
        out_shape=jax.ShapeDtypeStruct((HALF, 128), jnp.float32),
    )(TT, TT)


def _sc_vpass(v_row2d, n_row2d, v_off, n_off, V2):
    @functools.partial(
        pl.kernel,
        mesh=_SC_MESH,
        compiler_params=_SC_PARAMS,
        out_type=[
            jax.ShapeDtypeStruct((B, 128), jnp.float32),
            jax.ShapeDtypeStruct((B, 128), jnp.float32),
        ],
        scratch_types=[
            pltpu.VMEM((NCHUNK, C), jnp.int32),
            pltpu.VMEM((NCHUNK * NIDX_ROWS, 128), jnp.int32),
            pltpu.VMEM((BPW + L,), jnp.int32),
            pltpu.VMEM((BPW * NEG + L,), jnp.int32),
            pltpu.VMEM((C, 128), jnp.float32),
            pltpu.VMEM((C * NEG, 128), jnp.float32),
            pltpu.VMEM((C, 128), jnp.float32),
            pltpu.VMEM((C, 128), jnp.float32),
            pltpu.SemaphoreType.DMA,
        ],
    )
    def k(vr_hbm, nr_hbm, vo_hbm, no_hbm, v_hbm, ev_hbm, w_hbm,
          vrow, nrow, voffv, noffv, ev, nrows, evo, wo, sem):
        wid = lax.axis_index("s") * NC + lax.axis_index("c")
        base = wid * BPW
        pltpu.sync_copy(vr_hbm.at[pl.ds(wid * NCHUNK, NCHUNK)], vrow)
        nb = NCHUNK * NIDX_ROWS
        pltpu.sync_copy(nr_hbm.at[pl.ds(wid * nb, nb)], nrow)
        pltpu.sync_copy(vo_hbm.at[pl.ds(base, BPW)], voffv.at[pl.ds(0, BPW)])
        pltpu.sync_copy(no_hbm.at[pl.ds(base * NEG, BPW * NEG)],
                        noffv.at[pl.ds(0, BPW * NEG)])

        def chunk_body(c, _):
            copies = [pltpu.async_copy(v_hbm.at[vrow.at[c]], ev, sem)]
            for j in range(NIDX_ROWS):
                copies.append(pltpu.async_copy(
                    v_hbm.at[nrow.at[c * NIDX_ROWS + j]],
                    nrows.at[pl.ds(j * 128, 128)], sem))
            for cp in copies:
                cp.wait()

            def elem_body(i, _):
                vo = voffv[pl.ds(c * C + i, L)][0]
                for kk in range(4):
                    evo[i, pl.ds(kk * L, L)] = ev[i, pl.ds(vo + kk * L, L)]
                w = [None] * 4
                for n in range(NEG):
                    r = i * NEG + n
                    no = noffv[pl.ds(c * C * NEG + r, L)][0]
                    for kk in range(4):
                        x = nrows[r, pl.ds(no + kk * L, L)]
                        w[kk] = x if n == 0 else w[kk] + x
                for kk in range(4):
                    wo[i, pl.ds(kk * L, L)] = w[kk]
                return 0

            lax.fori_loop(0, C, elem_body, 0, unroll=False)
            pltpu.sync_copy(evo, ev_hbm.at[pl.ds(base + c * C, C)])
            pltpu.sync_copy(wo, w_hbm.at[pl.ds(base + c * C, C)])
            return 0

        lax.fori_loop(0, NCHUNK, chunk_body, 0, unroll=False)

    return k(v_row2d, n_row2d, v_off, n_off, V2)


def _sc_upass(u_row2d, u_off, U2, EV, W):
    @functools.partial(
        pl.kernel,
        mesh=_SC_MESH,
        compiler_params=_SC_PARAMS,
        out_type=[
            jax.ShapeDtypeStruct((B, L), jnp.float32),
            jax.ShapeDtypeStruct((B, L), jnp.float32),
        ],
        scratch_types=[
            pltpu.VMEM((NCHUNK, C), jnp.int32),
            pltpu.VMEM((BPW + L,), jnp.int32),
            pltpu.VMEM((C, 128), jnp.float32),
            pltpu.VMEM((C, 128), jnp.float32),
            pltpu.VMEM((C, 128), jnp.float32),
            pltpu.VMEM((BPW, L), jnp.float32),
            pltpu.VMEM((BPW, L), jnp.float32),
            pltpu.SemaphoreType.DMA,
        ],
    )
    def k(ur_hbm, uo_hbm, u_hbm, ev_hbm, w_hbm, p_hbm, q_hbm,
          urow, uoffv, eu, ev, wv, pw, qw, sem):
        wid = lax.axis_index("s") * NC + lax.axis_index("c")
        base = wid * BPW
        pltpu.sync_copy(ur_hbm.at[pl.ds(wid * NCHUNK, NCHUNK)], urow)
        pltpu.sync_copy(uo_hbm.at[pl.ds(base, BPW)], uoffv.at[pl.ds(0, BPW)])

        def chunk_body(c, _):
            cb = base + c * C
            copies = [
                pltpu.async_copy(u_hbm.at[urow.at[c]], eu, sem),
                pltpu.async_copy(ev_hbm.at[pl.ds(cb, C)], ev, sem),
                pltpu.async_copy(w_hbm.at[pl.ds(cb, C)], wv, sem),
            ]
            for cp in copies:
                cp.wait()

            def elem_body(i, _):
                uo = uoffv[pl.ds(c * C + i, L)][0]
                e = [eu[i, pl.ds(uo + kk * L, L)] for kk in range(4)]
                p = e[0] * ev[i, pl.ds(0, L)]
                q = e[0] * wv[i, pl.ds(0, L)]
                for kk in range(1, 4):
                    p = p + e[kk] * ev[i, pl.ds(kk * L, L)]
                    q = q + e[kk] * wv[i, pl.ds(kk * L, L)]
                pw[c * C + i, pl.ds(0, L)] = p
                qw[c * C + i, pl.ds(0, L)] = q
                return 0

            lax.fori_loop(0, C, elem_body, 0, unroll=False)
            return 0

        lax.fori_loop(0, NCHUNK, chunk_body, 0, unroll=False)
        pltpu.sync_copy(pw, p_hbm.at[pl.ds(base, BPW)])
        pltpu.sync_copy(qw, q_hbm.at[pl.ds(base, BPW)])

    return k(u_row2d, u_off, U2, EV, W)


def _tc_loss_body(p_ref, q_ref, o_ref):
    j = lax.broadcasted_iota(jnp.int32, (128, 8), 0)
    e = lax.broadcasted_iota(jnp.int32, (128, 8), 1)
    S = (j // L == e).astype(jnp.float32)
    a = jnp.dot(p_ref[...], S, preferred_element_type=jnp.float32)
    s = jnp.dot(q_ref[...], S, preferred_element_type=jnp.float32)

    def log_sigmoid(x):
        return jnp.minimum(x, 0.0) - jnp.log1p(jnp.exp(-jnp.abs(x)))

    ls = log_sigmoid(a) + log_sigmoid(-s)
    o_ref[...] = jnp.reshape(-jnp.sum(ls) / B, (1, 1))


def kernel(u_pos, v_pos, v_neg, U, V):
    u_pos = u_pos.astype(jnp.int32)
    v_pos = v_pos.astype(jnp.int32)
    n_flat = v_neg.astype(jnp.int32).reshape(-1)
    u_row2d = (u_pos & (HALF - 1)).reshape(B // C, C)
    v_row2d = (v_pos & (HALF - 1)).reshape(B // C, C)
    n_row2d = (n_flat & (HALF - 1)).reshape(B * NEG // 128, 128)
    u_off = (u_pos >> 19) << 6
    v_off = (v_pos >> 19) << 6
    n_off = (n_flat >> 19) << 6
    V2 = _repack(V.T)
    EV, W = _sc_vpass(v_row2d, n_row2d, v_off, n_off, V2)
    U2 = _repack(U.T)
    P, Q = _sc_upass(u_row2d, u_off, U2, EV, W)
    loss = pl.pallas_call(
        _tc_loss_body,
        out_shape=jax.ShapeDtypeStruct((1, 1), jnp.float32),
    )(P.reshape(B * L // 128, 128), Q.reshape(B * L // 128, 128))
    return loss[0, 0]
